# Initial kernel scaffold; baseline (speedup 1.0000x reference)
import jax
import jax.numpy as jnp
from jax.experimental import pallas as pl


def kernel(x, edge_index, pan_w1, lin_W1, lin_b1, pan_w2, lin_W2, lin_b2, cls_W, cls_b):
    raise NotImplementedError("write your pallas kernel here")


# speedup vs baseline: 6.7181x; 6.7181x over previous
import functools

import jax
import jax.numpy as jnp
import numpy as np
from jax import lax
from jax.experimental import pallas as pl
from jax.experimental.pallas import tpu as pltpu
from jax.experimental.pallas import tpu_sc as plsc

NC = 2
NS = 16
CH = 128


def _ceil_to(a, m):
    return (a + m - 1) // m * m


def _i32(v):
    return lax.convert_element_type(v, jnp.int32)


def _sc_degree(dstp, npad, n_chunks):
    rpt = npad // NS
    mesh = plsc.VectorSubcoreMesh(core_axis_name="c", subcore_axis_name="s")
    ep_w = n_chunks * CH

    @functools.partial(
        pl.kernel,
        out_type=jax.ShapeDtypeStruct((NC * npad, 128), jnp.float32),
        mesh=mesh,
        scratch_types=[
            pltpu.VMEM_SHARED((npad, 128), jnp.float32),
            pltpu.VMEM((CH, 128), jnp.float32),
            pltpu.VMEM((CH,), jnp.int32),
        ],
    )
    def k(dst_hbm, out_hbm, acc, buf, didx):
        c = _i32(lax.axis_index("c"))
        s = _i32(lax.axis_index("s"))
        wid = s * jnp.int32(NC) + c
        srow = s * jnp.int32(rpt)

        def zrow(i, _):
            for j in range(8):
                buf[i, pl.ds(j * 16, 16)] = jnp.zeros((16,), jnp.float32)
            return 0
        lax.fori_loop(jnp.int32(0), jnp.int32(CH), zrow, 0)
        for b in range(rpt // CH):
            pltpu.sync_copy(buf, acc.at[pl.ds(srow + jnp.int32(b * CH), CH)])
        plsc.subcore_barrier()

        def orow(i, _):
            for j in range(8):
                buf[i, pl.ds(j * 16, 16)] = jnp.ones((16,), jnp.float32)
            return 0
        lax.fori_loop(jnp.int32(0), jnp.int32(CH), orow, 0)

        def body(kk, _):
            base = wid * jnp.int32(ep_w) + kk * jnp.int32(CH)
            pltpu.sync_copy(dst_hbm.at[pl.ds(base, CH)], didx)
            pltpu.sync_copy(buf, acc.at[didx], add=True)
            return 0
        lax.fori_loop(jnp.int32(0), jnp.int32(n_chunks), body, 0)
        plsc.subcore_barrier()
        pltpu.sync_copy(acc.at[pl.ds(srow, rpt)],
                        out_hbm.at[pl.ds(c * jnp.int32(npad) + srow, rpt)])

    return k(dstp)


def _sc_spmm(xs, srcp, dstp, npad, n_chunks):
    d = xs.shape[1]
    rpt = npad // NS
    ep_w = n_chunks * CH
    mesh = plsc.VectorSubcoreMesh(core_axis_name="c", subcore_axis_name="s")

    @functools.partial(
        pl.kernel,
        out_type=jax.ShapeDtypeStruct((NC * npad, d), jnp.float32),
        mesh=mesh,
        scratch_types=[
            pltpu.VMEM_SHARED((npad, d), jnp.float32),
            pltpu.VMEM((CH, d), jnp.float32),
            pltpu.VMEM((CH,), jnp.int32),
            pltpu.VMEM((CH,), jnp.int32),
            pltpu.SemaphoreType.DMA,
        ],
    )
    def k(xs_hbm, src_hbm, dst_hbm, out_hbm, acc, rows, sidx, didx, sem):
        c = _i32(lax.axis_index("c"))
        s = _i32(lax.axis_index("s"))
        wid = s * jnp.int32(NC) + c
        srow = s * jnp.int32(rpt)

        def zrow(i, _):
            for j in range(d // 16):
                rows[i, pl.ds(j * 16, 16)] = jnp.zeros((16,), jnp.float32)
            return 0
        lax.fori_loop(jnp.int32(0), jnp.int32(CH), zrow, 0)
        for b in range(rpt // CH):
            pltpu.sync_copy(rows, acc.at[pl.ds(srow + jnp.int32(b * CH), CH)])
        plsc.subcore_barrier()

        def body(kk, _):
            base = wid * jnp.int32(ep_w) + kk * jnp.int32(CH)
            pltpu.sync_copy(src_hbm.at[pl.ds(base, CH)], sidx)
            pltpu.sync_copy(dst_hbm.at[pl.ds(base, CH)], didx)
            pltpu.async_copy(xs_hbm.at[sidx], rows, sem).wait()
            pltpu.sync_copy(rows, acc.at[didx], add=True)
            return 0
        lax.fori_loop(jnp.int32(0), jnp.int32(n_chunks), body, 0)
        plsc.subcore_barrier()
        pltpu.sync_copy(acc.at[pl.ds(srow, rpt)],
                        out_hbm.at[pl.ds(c * jnp.int32(npad) + srow, rpt)])

    return k(xs, srcp, dstp)


def _sc_edge_combine(ttab, srcp, dstp, ep_w):
    ep = srcp.shape[0]
    trows = ttab.shape[0]
    EC = 256
    n2 = ep_w // EC
    rows_w = ep_w * 4 // 128
    mesh = plsc.VectorSubcoreMesh(core_axis_name="c", subcore_axis_name="s")

    @functools.partial(
        pl.kernel,
        out_type=jax.ShapeDtypeStruct((ep * 4 // 128, 128), jnp.float32),
        mesh=mesh,
        compiler_params=pltpu.CompilerParams(needs_layout_passes=False),
        scratch_types=[
            pltpu.VMEM((trows, 128), jnp.float32),
            pltpu.VMEM((EC * 4 // 128, 128), jnp.float32),
            pltpu.VMEM((EC,), jnp.int32),
            pltpu.VMEM((EC,), jnp.int32),
        ],
    )
    def k(tab_hbm, src_hbm, dst_hbm, out_hbm, tbuf, obuf, sidx, didx):
        c = _i32(lax.axis_index("c"))
        s = _i32(lax.axis_index("s"))
        wid = s * jnp.int32(NC) + c
        pltpu.sync_copy(tab_hbm, tbuf)
        lane = lax.iota(jnp.int32, 16)

        def body(kk, _):
            base = wid * jnp.int32(ep_w) + kk * jnp.int32(EC)
            pltpu.sync_copy(src_hbm.at[pl.ds(base, EC)], sidx)
            pltpu.sync_copy(dst_hbm.at[pl.ds(base, EC)], didx)
            for j in range(EC // 16):
                sv = sidx[pl.ds(j * 16, 16)] * np.int32(8)
                dv = didx[pl.ds(j * 16, 16)] * np.int32(8) + np.int32(4)
                for cc in range(4):
                    fa = sv + np.int32(cc)
                    fb = dv + np.int32(cc)
                    a = plsc.load_gather(
                        tbuf, [lax.shift_right_logical(fa, np.int32(7)),
                               lax.bitwise_and(fa, np.int32(127))])
                    b = plsc.load_gather(
                        tbuf, [lax.shift_right_logical(fb, np.int32(7)),
                               lax.bitwise_and(fb, np.int32(127))])
                    fo = (lane + np.int32(j * 16)) * np.int32(4) + np.int32(cc)
                    plsc.store_scatter(
                        obuf, [lax.shift_right_logical(fo, np.int32(7)),
                               lax.bitwise_and(fo, np.int32(127))], a + b)
            rowbase = wid * jnp.int32(rows_w) + kk * jnp.int32(EC * 4 // 128)
            pltpu.sync_copy(obuf, out_hbm.at[pl.ds(rowbase, EC * 4 // 128)])
            return 0
        lax.fori_loop(jnp.int32(0), jnp.int32(n2), body, 0)

    return k(ttab, srcp, dstp)


_R = 256
_Z = np.int32(0)


def _tc_prescale(cvec, x, dp0, dp1):
    n = x.shape[0]
    grid = (pl.cdiv(n, _R),)

    def body(c_ref, x_ref, d0_ref, d1_ref,
             xs1_o, pe1_o, s1_o, dinv2_o, pe2_o, s2_o):
        deg_e = d0_ref[:, 0:1] + d1_ref[:, 0:1]
        d1 = c_ref[0]
        e1 = c_ref[1]
        d2 = c_ref[2]
        e2 = c_ref[3]
        g1 = d1 + e1 * deg_e
        g2 = d2 + e2 * deg_e
        i1 = jnp.where(g1 > 0, lax.rsqrt(g1), 0.0)
        i2 = jnp.where(g2 > 0, lax.rsqrt(g2), 0.0)
        xs1_o[...] = i1 * x_ref[...]
        pe1_o[...] = e1 * i1
        s1_o[...] = d1 * i1 * i1
        dinv2_o[...] = i2
        pe2_o[...] = e2 * i2
        s2_o[...] = d2 * i2 * i2

    v1 = jax.ShapeDtypeStruct((n, 1), jnp.float32)
    return pl.pallas_call(
        body,
        grid=grid,
        in_specs=[
            pl.BlockSpec((4,), lambda i: (_Z,), memory_space=pltpu.SMEM),
            pl.BlockSpec((_R, 128), lambda i: (i, _Z)),
            pl.BlockSpec((_R, 128), lambda i: (i, _Z)),
            pl.BlockSpec((_R, 128), lambda i: (i, _Z)),
        ],
        out_specs=[
            pl.BlockSpec((_R, 128), lambda i: (i, _Z)),
            pl.BlockSpec((_R, 1), lambda i: (i, _Z)),
            pl.BlockSpec((_R, 1), lambda i: (i, _Z)),
            pl.BlockSpec((_R, 1), lambda i: (i, _Z)),
            pl.BlockSpec((_R, 1), lambda i: (i, _Z)),
            pl.BlockSpec((_R, 1), lambda i: (i, _Z)),
        ],
        out_shape=(jax.ShapeDtypeStruct((n, 128), jnp.float32),
                   v1, v1, v1, v1, v1),
    )(cvec, x, dp0, dp1)


def _tc_layer1(x, p0, p1, pe1, s1, dinv2, w1t, b1):
    n = x.shape[0]
    grid = (pl.cdiv(n, _R),)

    def body(x_ref, p0_ref, p1_ref, pe_ref, s_ref, di_ref, w_ref, b_ref,
             h_o, xs_o):
        m = pe_ref[...] * (p0_ref[...] + p1_ref[...]) + s_ref[...] * x_ref[...]
        h = jnp.dot(m, w_ref[...], preferred_element_type=jnp.float32)
        h = jnp.maximum(h + b_ref[...], 0.0)
        h_o[...] = h
        xs_o[...] = di_ref[...] * h

    rb = pl.BlockSpec((_R, 128), lambda i: (i, _Z))
    vb = pl.BlockSpec((_R, 1), lambda i: (i, _Z))
    return pl.pallas_call(
        body,
        grid=grid,
        in_specs=[rb, rb, rb, vb, vb, vb,
                  pl.BlockSpec((128, 128), lambda i: (_Z, _Z)),
                  pl.BlockSpec((1, 128), lambda i: (_Z, _Z))],
        out_specs=[rb, rb],
        out_shape=(jax.ShapeDtypeStruct((n, 128), jnp.float32),
                   jax.ShapeDtypeStruct((n, 128), jnp.float32)),
    )(x, p0, p1, pe1, s1, dinv2, w1t, b1)


def _tc_layer2_cls(h1, p0, p1, pe2, s2, w2t, b2, wct, bc):
    n = h1.shape[0]
    grid = (pl.cdiv(n, _R),)

    def body(h1_ref, p0_ref, p1_ref, pe_ref, s_ref, w_ref, b_ref,
             wc_ref, bc_ref, tab_o):
        m = (pe_ref[...] * (p0_ref[...] + p1_ref[...])
             + s_ref[...] * h1_ref[...])
        h2 = jnp.dot(m, w_ref[...], preferred_element_type=jnp.float32)
        h2 = h2 + b_ref[...]
        tab_o[...] = jnp.dot(h2, wc_ref[...],
                             preferred_element_type=jnp.float32) + bc_ref[...]

    rb = pl.BlockSpec((_R, 128), lambda i: (i, _Z))
    vb = pl.BlockSpec((_R, 1), lambda i: (i, _Z))
    return pl.pallas_call(
        body,
        grid=grid,
        in_specs=[rb, rb, rb, vb, vb,
                  pl.BlockSpec((128, 128), lambda i: (_Z, _Z)),
                  pl.BlockSpec((1, 128), lambda i: (_Z, _Z)),
                  pl.BlockSpec((128, 8), lambda i: (_Z, _Z)),
                  pl.BlockSpec((1, 8), lambda i: (_Z, _Z))],
        out_specs=pl.BlockSpec((_R, 8), lambda i: (i, _Z)),
        out_shape=jax.ShapeDtypeStruct((n, 8), jnp.float32),
    )(h1, p0, p1, pe2, s2, w2t, b2, wct, bc)


def _impl(x, edge_index, pan_w1, lin_W1, lin_b1, pan_w2, lin_W2, lin_b2,
          cls_W, cls_b):
    n = x.shape[0]
    e = edge_index.shape[1]
    hid = lin_W1.shape[0]
    x = x.astype(jnp.float32)

    nw = NC * NS
    ep_w = _ceil_to(-(-e // nw), 256)
    ep = ep_w * nw
    n_chunks = ep_w // CH
    rpt = _ceil_to(-(-(n + 1) // NS), CH)
    npad = rpt * NS

    src = edge_index[0].astype(jnp.int32)
    dst = edge_index[1].astype(jnp.int32)
    pad = ep - e
    srcp = jnp.concatenate([src, jnp.zeros((pad,), jnp.int32)])
    dst_sc = jnp.concatenate([dst, jnp.full((pad,), n, jnp.int32)])
    dst_ga = jnp.concatenate([dst, jnp.zeros((pad,), jnp.int32)])

    d1 = pan_w1[0].astype(jnp.float32)
    e1 = (pan_w1[0] * pan_w1[1]).astype(jnp.float32)
    d2 = pan_w2[0].astype(jnp.float32)
    e2 = (pan_w2[0] * pan_w2[1]).astype(jnp.float32)
    cvec = jnp.stack([d1, e1, d2, e2])

    degp = _sc_degree(dst_sc, npad, n_chunks)
    dp0 = lax.slice(degp, (0, 0), (n, 128))
    dp1 = lax.slice(degp, (npad, 0), (npad + n, 128))

    xs1, pe1, s1, dinv2, pe2, s2 = _tc_prescale(cvec, x, dp0, dp1)

    agg1 = _sc_spmm(xs1, srcp, dst_sc, npad, n_chunks)
    a10 = lax.slice(agg1, (0, 0), (n, hid))
    a11 = lax.slice(agg1, (npad, 0), (npad + n, hid))

    w1t = lin_W1.astype(jnp.float32).T
    b1 = lin_b1.astype(jnp.float32).reshape(1, hid)
    h1, xs2 = _tc_layer1(x, a10, a11, pe1, s1, dinv2, w1t, b1)

    agg2 = _sc_spmm(xs2, srcp, dst_sc, npad, n_chunks)
    a20 = lax.slice(agg2, (0, 0), (n, hid))
    a21 = lax.slice(agg2, (npad, 0), (npad + n, hid))

    w2t = lin_W2.astype(jnp.float32).T
    b2 = lin_b2.astype(jnp.float32).reshape(1, hid)
    out_dim = cls_W.shape[0]
    wa = lax.slice(cls_W, (0, 0), (out_dim, hid))
    wb = lax.slice(cls_W, (0, hid), (out_dim, 2 * hid))
    wct = jnp.concatenate([wa, wb], axis=0).astype(jnp.float32).T
    bc = jnp.concatenate([jnp.zeros((out_dim,), jnp.float32),
                          cls_b.astype(jnp.float32)]).reshape(1, 2 * out_dim)
    tab = _tc_layer2_cls(h1, a20, a21, pe2, s2, w2t, b2, wct, bc)

    npack = _ceil_to(n, 16)
    if npack != n:
        tab = jnp.concatenate(
            [tab, jnp.zeros((npack - n, 2 * out_dim), jnp.float32)])
    ttab = tab.reshape(npack * 8 // 128, 128)
    o = _sc_edge_combine(ttab, srcp, dst_ga, ep_w)
    return o.reshape(ep, 4)[:e]


kernel = jax.jit(_impl)


# speedup vs baseline: 10.5300x; 1.5674x over previous
import functools

import jax
import jax.numpy as jnp
import numpy as np
from jax import lax
from jax.experimental import pallas as pl
from jax.experimental.pallas import tpu as pltpu
from jax.experimental.pallas import tpu_sc as plsc

NC = 2
NS = 16
CH = 128


def _ceil_to(a, m):
    return (a + m - 1) // m * m


def _i32(v):
    return lax.convert_element_type(v, jnp.int32)


def _sc_degree(dstp, npad, n_chunks):
    rpt = npad // NS
    mesh = plsc.VectorSubcoreMesh(core_axis_name="c", subcore_axis_name="s")
    ep_w = n_chunks * CH

    @functools.partial(
        pl.kernel,
        out_type=jax.ShapeDtypeStruct((NC * npad, 128), jnp.float32),
        mesh=mesh,
        scratch_types=[
            pltpu.VMEM_SHARED((npad, 128), jnp.float32),
            pltpu.VMEM((CH, 128), jnp.float32),
            pltpu.VMEM((CH,), jnp.int32),
        ],
    )
    def k(dst_hbm, out_hbm, acc, buf, didx):
        c = _i32(lax.axis_index("c"))
        s = _i32(lax.axis_index("s"))
        wid = s * jnp.int32(NC) + c
        srow = s * jnp.int32(rpt)

        def zrow(i, _):
            for j in range(8):
                buf[i, pl.ds(j * 16, 16)] = jnp.zeros((16,), jnp.float32)
            return 0
        lax.fori_loop(jnp.int32(0), jnp.int32(CH), zrow, 0)
        for b in range(rpt // CH):
            pltpu.sync_copy(buf, acc.at[pl.ds(srow + jnp.int32(b * CH), CH)])
        plsc.subcore_barrier()

        def orow(i, _):
            for j in range(8):
                buf[i, pl.ds(j * 16, 16)] = jnp.ones((16,), jnp.float32)
            return 0
        lax.fori_loop(jnp.int32(0), jnp.int32(CH), orow, 0)

        def body(kk, _):
            base = wid * jnp.int32(ep_w) + kk * jnp.int32(CH)
            pltpu.sync_copy(dst_hbm.at[pl.ds(base, CH)], didx)
            pltpu.sync_copy(buf, acc.at[didx], add=True)
            return 0
        lax.fori_loop(jnp.int32(0), jnp.int32(n_chunks), body, 0)
        plsc.subcore_barrier()
        pltpu.sync_copy(acc.at[pl.ds(srow, rpt)],
                        out_hbm.at[pl.ds(c * jnp.int32(npad) + srow, rpt)])

    return k(dstp)


def _sc_spmm(xs, srcp, dstp, npad, n_chunks):
    d = xs.shape[1]
    rpt = npad // NS
    ep_w = n_chunks * CH
    mesh = plsc.VectorSubcoreMesh(core_axis_name="c", subcore_axis_name="s")

    @functools.partial(
        pl.kernel,
        out_type=jax.ShapeDtypeStruct((NC * npad, d), jnp.float32),
        mesh=mesh,
        scratch_types=[
            pltpu.VMEM_SHARED((npad, d), jnp.float32),
            pltpu.VMEM((CH, d), jnp.float32),
            pltpu.VMEM((CH,), jnp.int32),
            pltpu.VMEM((CH,), jnp.int32),
            pltpu.SemaphoreType.DMA,
        ],
    )
    def k(xs_hbm, src_hbm, dst_hbm, out_hbm, acc, rows, sidx, didx, sem):
        c = _i32(lax.axis_index("c"))
        s = _i32(lax.axis_index("s"))
        wid = s * jnp.int32(NC) + c
        srow = s * jnp.int32(rpt)

        def zrow(i, _):
            for j in range(d // 16):
                rows[i, pl.ds(j * 16, 16)] = jnp.zeros((16,), jnp.float32)
            return 0
        lax.fori_loop(jnp.int32(0), jnp.int32(CH), zrow, 0)
        for b in range(rpt // CH):
            pltpu.sync_copy(rows, acc.at[pl.ds(srow + jnp.int32(b * CH), CH)])
        plsc.subcore_barrier()

        def body(kk, _):
            base = wid * jnp.int32(ep_w) + kk * jnp.int32(CH)
            pltpu.sync_copy(src_hbm.at[pl.ds(base, CH)], sidx)
            pltpu.sync_copy(dst_hbm.at[pl.ds(base, CH)], didx)
            pltpu.async_copy(xs_hbm.at[sidx], rows, sem).wait()
            pltpu.sync_copy(rows, acc.at[didx], add=True)
            return 0
        lax.fori_loop(jnp.int32(0), jnp.int32(n_chunks), body, 0)
        plsc.subcore_barrier()
        pltpu.sync_copy(acc.at[pl.ds(srow, rpt)],
                        out_hbm.at[pl.ds(c * jnp.int32(npad) + srow, rpt)])

    return k(xs, srcp, dstp)


def _sc_edge_combine(ttab, srcp, dstp, ep_w):
    ep = srcp.shape[0]
    trows = ttab.shape[0]
    EC = 256
    n2 = ep_w // EC
    rows_w = ep_w * 4 // 128
    mesh = plsc.VectorSubcoreMesh(core_axis_name="c", subcore_axis_name="s")

    @functools.partial(
        pl.kernel,
        out_type=jax.ShapeDtypeStruct((ep * 4 // 128, 128), jnp.float32),
        mesh=mesh,
        compiler_params=pltpu.CompilerParams(needs_layout_passes=False),
        scratch_types=[
            pltpu.VMEM((trows, 128), jnp.float32),
            pltpu.VMEM((EC * 4 // 128, 128), jnp.float32),
            pltpu.VMEM((EC,), jnp.int32),
            pltpu.VMEM((EC,), jnp.int32),
        ],
    )
    def k(tab_hbm, src_hbm, dst_hbm, out_hbm, tbuf, obuf, sidx, didx):
        c = _i32(lax.axis_index("c"))
        s = _i32(lax.axis_index("s"))
        wid = s * jnp.int32(NC) + c
        pltpu.sync_copy(tab_hbm, tbuf)
        lane = lax.iota(jnp.int32, 16)

        def body(kk, _):
            base = wid * jnp.int32(ep_w) + kk * jnp.int32(EC)
            pltpu.sync_copy(src_hbm.at[pl.ds(base, EC)], sidx)
            pltpu.sync_copy(dst_hbm.at[pl.ds(base, EC)], didx)
            for j in range(EC // 16):
                sv = sidx[pl.ds(j * 16, 16)] * np.int32(8)
                dv = didx[pl.ds(j * 16, 16)] * np.int32(8) + np.int32(4)
                for cc in range(4):
                    fa = sv + np.int32(cc)
                    fb = dv + np.int32(cc)
                    a = plsc.load_gather(
                        tbuf, [lax.shift_right_logical(fa, np.int32(7)),
                               lax.bitwise_and(fa, np.int32(127))])
                    b = plsc.load_gather(
                        tbuf, [lax.shift_right_logical(fb, np.int32(7)),
                               lax.bitwise_and(fb, np.int32(127))])
                    fo = (lane + np.int32(j * 16)) * np.int32(4) + np.int32(cc)
                    plsc.store_scatter(
                        obuf, [lax.shift_right_logical(fo, np.int32(7)),
                               lax.bitwise_and(fo, np.int32(127))], a + b)
            rowbase = wid * jnp.int32(rows_w) + kk * jnp.int32(EC * 4 // 128)
            pltpu.sync_copy(obuf, out_hbm.at[pl.ds(rowbase, EC * 4 // 128)])
            return 0
        lax.fori_loop(jnp.int32(0), jnp.int32(n2), body, 0)

    return k(ttab, srcp, dstp)


_R = 256
_Z = np.int32(0)


def _tc_prescale(cvec, x, dp0, dp1):
    n = x.shape[0]
    grid = (pl.cdiv(n, _R),)

    def body(c_ref, x_ref, d0_ref, d1_ref,
             xs1_o, pe1_o, s1_o, dinv2_o, pe2_o, s2_o):
        deg_e = d0_ref[:, 0:1] + d1_ref[:, 0:1]
        d1 = c_ref[0]
        e1 = c_ref[1]
        d2 = c_ref[2]
        e2 = c_ref[3]
        g1 = d1 + e1 * deg_e
        g2 = d2 + e2 * deg_e
        i1 = jnp.where(g1 > 0, lax.rsqrt(g1), 0.0)
        i2 = jnp.where(g2 > 0, lax.rsqrt(g2), 0.0)
        xs1_o[...] = i1 * x_ref[...]
        pe1_o[...] = e1 * i1
        s1_o[...] = d1 * i1 * i1
        dinv2_o[...] = i2
        pe2_o[...] = e2 * i2
        s2_o[...] = d2 * i2 * i2

    v1 = jax.ShapeDtypeStruct((n, 1), jnp.float32)
    return pl.pallas_call(
        body,
        grid=grid,
        in_specs=[
            pl.BlockSpec((4,), lambda i: (_Z,), memory_space=pltpu.SMEM),
            pl.BlockSpec((_R, 128), lambda i: (i, _Z)),
            pl.BlockSpec((_R, 128), lambda i: (i, _Z)),
            pl.BlockSpec((_R, 128), lambda i: (i, _Z)),
        ],
        out_specs=[
            pl.BlockSpec((_R, 128), lambda i: (i, _Z)),
            pl.BlockSpec((_R, 1), lambda i: (i, _Z)),
            pl.BlockSpec((_R, 1), lambda i: (i, _Z)),
            pl.BlockSpec((_R, 1), lambda i: (i, _Z)),
            pl.BlockSpec((_R, 1), lambda i: (i, _Z)),
            pl.BlockSpec((_R, 1), lambda i: (i, _Z)),
        ],
        out_shape=(jax.ShapeDtypeStruct((n, 128), jnp.float32),
                   v1, v1, v1, v1, v1),
    )(cvec, x, dp0, dp1)


def _tc_layer1(x, p0, p1, pe1, s1, dinv2, w1t, b1):
    n = x.shape[0]
    grid = (pl.cdiv(n, _R),)

    def body(x_ref, p0_ref, p1_ref, pe_ref, s_ref, di_ref, w_ref, b_ref,
             h_o, xs_o):
        m = pe_ref[...] * (p0_ref[...] + p1_ref[...]) + s_ref[...] * x_ref[...]
        h = jnp.dot(m, w_ref[...], preferred_element_type=jnp.float32)
        h = jnp.maximum(h + b_ref[...], 0.0)
        h_o[...] = h
        xs_o[...] = di_ref[...] * h

    rb = pl.BlockSpec((_R, 128), lambda i: (i, _Z))
    vb = pl.BlockSpec((_R, 1), lambda i: (i, _Z))
    return pl.pallas_call(
        body,
        grid=grid,
        in_specs=[rb, rb, rb, vb, vb, vb,
                  pl.BlockSpec((128, 128), lambda i: (_Z, _Z)),
                  pl.BlockSpec((1, 128), lambda i: (_Z, _Z))],
        out_specs=[rb, rb],
        out_shape=(jax.ShapeDtypeStruct((n, 128), jnp.float32),
                   jax.ShapeDtypeStruct((n, 128), jnp.float32)),
    )(x, p0, p1, pe1, s1, dinv2, w1t, b1)


def _tc_layer2_cls(h1, p0, p1, pe2, s2, w2t, b2, wct, bc):
    n = h1.shape[0]
    grid = (pl.cdiv(n, _R),)

    def body(h1_ref, p0_ref, p1_ref, pe_ref, s_ref, w_ref, b_ref,
             wc_ref, bc_ref, tab_o):
        m = (pe_ref[...] * (p0_ref[...] + p1_ref[...])
             + s_ref[...] * h1_ref[...])
        h2 = jnp.dot(m, w_ref[...], preferred_element_type=jnp.float32)
        h2 = h2 + b_ref[...]
        tab_o[...] = jnp.dot(h2, wc_ref[...],
                             preferred_element_type=jnp.float32) + bc_ref[...]

    rb = pl.BlockSpec((_R, 128), lambda i: (i, _Z))
    vb = pl.BlockSpec((_R, 1), lambda i: (i, _Z))
    return pl.pallas_call(
        body,
        grid=grid,
        in_specs=[rb, rb, rb, vb, vb,
                  pl.BlockSpec((128, 128), lambda i: (_Z, _Z)),
                  pl.BlockSpec((1, 128), lambda i: (_Z, _Z)),
                  pl.BlockSpec((128, 8), lambda i: (_Z, _Z)),
                  pl.BlockSpec((1, 8), lambda i: (_Z, _Z))],
        out_specs=pl.BlockSpec((_R, 8), lambda i: (i, _Z)),
        out_shape=jax.ShapeDtypeStruct((n, 8), jnp.float32),
    )(h1, p0, p1, pe2, s2, w2t, b2, wct, bc)


def _impl(x, edge_index, pan_w1, lin_W1, lin_b1, pan_w2, lin_W2, lin_b2,
          cls_W, cls_b):
    n = x.shape[0]
    e = edge_index.shape[1]
    hid = lin_W1.shape[0]
    x = x.astype(jnp.float32)

    nw = NC * NS
    ep_w = _ceil_to(-(-e // nw), 256)
    ep = ep_w * nw
    n_chunks = ep_w // CH
    rpt = _ceil_to(-(-(n + 1) // NS), CH)
    npad = rpt * NS

    src = edge_index[0].astype(jnp.int32)
    dst = edge_index[1].astype(jnp.int32)
    pad = ep - e
    pidx = lax.iota(jnp.int32, pad)
    spare = npad - n
    srcp = jnp.concatenate([src, pidx % jnp.int32(n)])
    dst_sc = jnp.concatenate([dst, jnp.int32(n) + pidx % jnp.int32(spare)])
    dst_ga = jnp.concatenate([dst, jnp.zeros((pad,), jnp.int32)])

    d1 = pan_w1[0].astype(jnp.float32)
    e1 = (pan_w1[0] * pan_w1[1]).astype(jnp.float32)
    d2 = pan_w2[0].astype(jnp.float32)
    e2 = (pan_w2[0] * pan_w2[1]).astype(jnp.float32)
    cvec = jnp.stack([d1, e1, d2, e2])

    degp = _sc_degree(dst_sc, npad, n_chunks)
    dp0 = lax.slice(degp, (0, 0), (n, 128))
    dp1 = lax.slice(degp, (npad, 0), (npad + n, 128))

    xs1, pe1, s1, dinv2, pe2, s2 = _tc_prescale(cvec, x, dp0, dp1)

    agg1 = _sc_spmm(xs1, srcp, dst_sc, npad, n_chunks)
    a10 = lax.slice(agg1, (0, 0), (n, hid))
    a11 = lax.slice(agg1, (npad, 0), (npad + n, hid))

    w1t = lin_W1.astype(jnp.float32).T
    b1 = lin_b1.astype(jnp.float32).reshape(1, hid)
    h1, xs2 = _tc_layer1(x, a10, a11, pe1, s1, dinv2, w1t, b1)

    agg2 = _sc_spmm(xs2, srcp, dst_sc, npad, n_chunks)
    a20 = lax.slice(agg2, (0, 0), (n, hid))
    a21 = lax.slice(agg2, (npad, 0), (npad + n, hid))

    w2t = lin_W2.astype(jnp.float32).T
    b2 = lin_b2.astype(jnp.float32).reshape(1, hid)
    out_dim = cls_W.shape[0]
    wa = lax.slice(cls_W, (0, 0), (out_dim, hid))
    wb = lax.slice(cls_W, (0, hid), (out_dim, 2 * hid))
    wct = jnp.concatenate([wa, wb], axis=0).astype(jnp.float32).T
    bc = jnp.concatenate([jnp.zeros((out_dim,), jnp.float32),
                          cls_b.astype(jnp.float32)]).reshape(1, 2 * out_dim)
    tab = _tc_layer2_cls(h1, a20, a21, pe2, s2, w2t, b2, wct, bc)

    npack = _ceil_to(n, 16)
    if npack != n:
        tab = jnp.concatenate(
            [tab, jnp.zeros((npack - n, 2 * out_dim), jnp.float32)])
    ttab = tab.reshape(npack * 8 // 128, 128)
    o = _sc_edge_combine(ttab, srcp, dst_ga, ep_w)
    return o.reshape(ep, 4)[:e]


kernel = jax.jit(_impl)


# speedup vs baseline: 12.8814x; 1.2233x over previous
import functools

import jax
import jax.numpy as jnp
import numpy as np
from jax import lax
from jax.experimental import pallas as pl
from jax.experimental.pallas import tpu as pltpu
from jax.experimental.pallas import tpu_sc as plsc

NC = 2
NS = 16
CH = 128


def _ceil_to(a, m):
    return (a + m - 1) // m * m


def _i32(v):
    return lax.convert_element_type(v, jnp.int32)


def _sc_degree(dstp, npad, n_chunks):
    rpt = npad // NS
    mesh = plsc.VectorSubcoreMesh(core_axis_name="c", subcore_axis_name="s")
    ep_w = n_chunks * CH

    @functools.partial(
        pl.kernel,
        out_type=jax.ShapeDtypeStruct((NC * npad, 128), jnp.float32),
        mesh=mesh,
        scratch_types=[
            pltpu.VMEM_SHARED((npad, 128), jnp.float32),
            pltpu.VMEM((CH, 128), jnp.float32),
            pltpu.VMEM((CH,), jnp.int32),
        ],
    )
    def k(dst_hbm, out_hbm, acc, buf, didx):
        c = _i32(lax.axis_index("c"))
        s = _i32(lax.axis_index("s"))
        wid = s * jnp.int32(NC) + c
        srow = s * jnp.int32(rpt)

        def zrow(i, _):
            for j in range(8):
                buf[i, pl.ds(j * 16, 16)] = jnp.zeros((16,), jnp.float32)
            return 0
        lax.fori_loop(jnp.int32(0), jnp.int32(CH), zrow, 0)
        for b in range(rpt // CH):
            pltpu.sync_copy(buf, acc.at[pl.ds(srow + jnp.int32(b * CH), CH)])
        plsc.subcore_barrier()

        def orow(i, _):
            for j in range(8):
                buf[i, pl.ds(j * 16, 16)] = jnp.ones((16,), jnp.float32)
            return 0
        lax.fori_loop(jnp.int32(0), jnp.int32(CH), orow, 0)

        def body(kk, _):
            base = wid * jnp.int32(ep_w) + kk * jnp.int32(CH)
            pltpu.sync_copy(dst_hbm.at[pl.ds(base, CH)], didx)
            pltpu.sync_copy(buf, acc.at[didx], add=True)
            return 0
        lax.fori_loop(jnp.int32(0), jnp.int32(n_chunks), body, 0)
        plsc.subcore_barrier()
        pltpu.sync_copy(acc.at[pl.ds(srow, rpt)],
                        out_hbm.at[pl.ds(c * jnp.int32(npad) + srow, rpt)])

    return k(dstp)


def _sc_spmm(xs, srcp, dstp, npad, n_chunks):
    d = xs.shape[1]
    rpt = npad // NS
    ep_w = n_chunks * CH
    NB = 2
    assert n_chunks % NB == 0
    mesh = plsc.VectorSubcoreMesh(core_axis_name="c", subcore_axis_name="s")

    @functools.partial(
        pl.kernel,
        out_type=jax.ShapeDtypeStruct((NC * npad, d), jnp.float32),
        mesh=mesh,
        scratch_types=(
            [pltpu.VMEM_SHARED((npad, d), jnp.float32)]
            + [pltpu.VMEM((CH, d), jnp.float32)] * NB
            + [pltpu.VMEM((CH,), jnp.int32)] * (2 * NB)
            + [pltpu.SemaphoreType.DMA] * NB
        ),
    )
    def k(xs_hbm, src_hbm, dst_hbm, out_hbm, acc, *scr):
        rows = scr[0:NB]
        sidx = scr[NB:2 * NB]
        didx = scr[2 * NB:3 * NB]
        sems = scr[3 * NB:4 * NB]
        c = _i32(lax.axis_index("c"))
        s = _i32(lax.axis_index("s"))
        wid = s * jnp.int32(NC) + c
        srow = s * jnp.int32(rpt)

        def zrow(i, _):
            for j in range(d // 16):
                rows[0][i, pl.ds(j * 16, 16)] = jnp.zeros((16,), jnp.float32)
            return 0
        lax.fori_loop(jnp.int32(0), jnp.int32(CH), zrow, 0)
        for b in range(rpt // CH):
            pltpu.sync_copy(rows[0],
                            acc.at[pl.ds(srow + jnp.int32(b * CH), CH)])
        plsc.subcore_barrier()

        for b in range(NB):
            base = wid * jnp.int32(ep_w) + jnp.int32(b * CH)
            pltpu.sync_copy(src_hbm.at[pl.ds(base, CH)], sidx[b])
            pltpu.sync_copy(dst_hbm.at[pl.ds(base, CH)], didx[b])
            pltpu.async_copy(xs_hbm.at[sidx[b]], rows[b], sems[b])

        def body(g, _):
            for b in range(NB):
                kk = g * jnp.int32(NB) + jnp.int32(b)
                pltpu.make_async_copy(xs_hbm.at[sidx[b]], rows[b],
                                      sems[b]).wait()
                pltpu.sync_copy(rows[b], acc.at[didx[b]], add=True)
                kn = kk + jnp.int32(NB)

                @pl.when(kn < jnp.int32(n_chunks))
                def _():
                    base = wid * jnp.int32(ep_w) + kn * jnp.int32(CH)
                    pltpu.sync_copy(src_hbm.at[pl.ds(base, CH)], sidx[b])
                    pltpu.sync_copy(dst_hbm.at[pl.ds(base, CH)], didx[b])
                    pltpu.async_copy(xs_hbm.at[sidx[b]], rows[b], sems[b])
            return 0
        lax.fori_loop(jnp.int32(0), jnp.int32(n_chunks // NB), body, 0)
        plsc.subcore_barrier()
        pltpu.sync_copy(acc.at[pl.ds(srow, rpt)],
                        out_hbm.at[pl.ds(c * jnp.int32(npad) + srow, rpt)])

    return k(xs, srcp, dstp)


def _sc_edge_combine(ttab, srcp, dstp, ep_w):
    ep = srcp.shape[0]
    trows = ttab.shape[0]
    EC = 256
    n2 = ep_w // EC
    rows_w = ep_w * 4 // 128
    mesh = plsc.VectorSubcoreMesh(core_axis_name="c", subcore_axis_name="s")

    @functools.partial(
        pl.kernel,
        out_type=jax.ShapeDtypeStruct((ep * 4 // 128, 128), jnp.float32),
        mesh=mesh,
        compiler_params=pltpu.CompilerParams(needs_layout_passes=False),
        scratch_types=[
            pltpu.VMEM((trows, 128), jnp.float32),
            pltpu.VMEM((EC * 4 // 128, 128), jnp.float32),
            pltpu.VMEM((EC,), jnp.int32),
            pltpu.VMEM((EC,), jnp.int32),
        ],
    )
    def k(tab_hbm, src_hbm, dst_hbm, out_hbm, tbuf, obuf, sidx, didx):
        c = _i32(lax.axis_index("c"))
        s = _i32(lax.axis_index("s"))
        wid = s * jnp.int32(NC) + c
        pltpu.sync_copy(tab_hbm, tbuf)
        lane = lax.iota(jnp.int32, 16)

        def body(kk, _):
            base = wid * jnp.int32(ep_w) + kk * jnp.int32(EC)
            pltpu.sync_copy(src_hbm.at[pl.ds(base, EC)], sidx)
            pltpu.sync_copy(dst_hbm.at[pl.ds(base, EC)], didx)
            for j in range(EC // 16):
                sv = sidx[pl.ds(j * 16, 16)] * np.int32(8)
                dv = didx[pl.ds(j * 16, 16)] * np.int32(8) + np.int32(4)
                for cc in range(4):
                    fa = sv + np.int32(cc)
                    fb = dv + np.int32(cc)
                    a = plsc.load_gather(
                        tbuf, [lax.shift_right_logical(fa, np.int32(7)),
                               lax.bitwise_and(fa, np.int32(127))])
                    b = plsc.load_gather(
                        tbuf, [lax.shift_right_logical(fb, np.int32(7)),
                               lax.bitwise_and(fb, np.int32(127))])
                    fo = (lane + np.int32(j * 16)) * np.int32(4) + np.int32(cc)
                    plsc.store_scatter(
                        obuf, [lax.shift_right_logical(fo, np.int32(7)),
                               lax.bitwise_and(fo, np.int32(127))], a + b)
            rowbase = wid * jnp.int32(rows_w) + kk * jnp.int32(EC * 4 // 128)
            pltpu.sync_copy(obuf, out_hbm.at[pl.ds(rowbase, EC * 4 // 128)])
            return 0
        lax.fori_loop(jnp.int32(0), jnp.int32(n2), body, 0)

    return k(ttab, srcp, dstp)


_R = 256
_Z = np.int32(0)


def _tc_prescale(cvec, x, dp0, dp1):
    n = x.shape[0]
    grid = (pl.cdiv(n, _R),)

    def body(c_ref, x_ref, d0_ref, d1_ref,
             xs1_o, pe1_o, s1_o, dinv2_o, pe2_o, s2_o):
        deg_e = d0_ref[:, 0:1] + d1_ref[:, 0:1]
        d1 = c_ref[0]
        e1 = c_ref[1]
        d2 = c_ref[2]
        e2 = c_ref[3]
        g1 = d1 + e1 * deg_e
        g2 = d2 + e2 * deg_e
        i1 = jnp.where(g1 > 0, lax.rsqrt(g1), 0.0)
        i2 = jnp.where(g2 > 0, lax.rsqrt(g2), 0.0)
        xs1_o[...] = i1 * x_ref[...]
        pe1_o[...] = e1 * i1
        s1_o[...] = d1 * i1 * i1
        dinv2_o[...] = i2
        pe2_o[...] = e2 * i2
        s2_o[...] = d2 * i2 * i2

    v1 = jax.ShapeDtypeStruct((n, 1), jnp.float32)
    return pl.pallas_call(
        body,
        grid=grid,
        in_specs=[
            pl.BlockSpec((4,), lambda i: (_Z,), memory_space=pltpu.SMEM),
            pl.BlockSpec((_R, 128), lambda i: (i, _Z)),
            pl.BlockSpec((_R, 128), lambda i: (i, _Z)),
            pl.BlockSpec((_R, 128), lambda i: (i, _Z)),
        ],
        out_specs=[
            pl.BlockSpec((_R, 128), lambda i: (i, _Z)),
            pl.BlockSpec((_R, 1), lambda i: (i, _Z)),
            pl.BlockSpec((_R, 1), lambda i: (i, _Z)),
            pl.BlockSpec((_R, 1), lambda i: (i, _Z)),
            pl.BlockSpec((_R, 1), lambda i: (i, _Z)),
            pl.BlockSpec((_R, 1), lambda i: (i, _Z)),
        ],
        out_shape=(jax.ShapeDtypeStruct((n, 128), jnp.float32),
                   v1, v1, v1, v1, v1),
    )(cvec, x, dp0, dp1)


def _tc_layer1(x, p0, p1, pe1, s1, dinv2, w1t, b1):
    n = x.shape[0]
    grid = (pl.cdiv(n, _R),)

    def body(x_ref, p0_ref, p1_ref, pe_ref, s_ref, di_ref, w_ref, b_ref,
             h_o, xs_o):
        m = pe_ref[...] * (p0_ref[...] + p1_ref[...]) + s_ref[...] * x_ref[...]
        h = jnp.dot(m, w_ref[...], preferred_element_type=jnp.float32)
        h = jnp.maximum(h + b_ref[...], 0.0)
        h_o[...] = h
        xs_o[...] = di_ref[...] * h

    rb = pl.BlockSpec((_R, 128), lambda i: (i, _Z))
    vb = pl.BlockSpec((_R, 1), lambda i: (i, _Z))
    return pl.pallas_call(
        body,
        grid=grid,
        in_specs=[rb, rb, rb, vb, vb, vb,
                  pl.BlockSpec((128, 128), lambda i: (_Z, _Z)),
                  pl.BlockSpec((1, 128), lambda i: (_Z, _Z))],
        out_specs=[rb, rb],
        out_shape=(jax.ShapeDtypeStruct((n, 128), jnp.float32),
                   jax.ShapeDtypeStruct((n, 128), jnp.float32)),
    )(x, p0, p1, pe1, s1, dinv2, w1t, b1)


def _tc_layer2_cls(h1, p0, p1, pe2, s2, w2t, b2, wct, bc):
    n = h1.shape[0]
    grid = (pl.cdiv(n, _R),)

    def body(h1_ref, p0_ref, p1_ref, pe_ref, s_ref, w_ref, b_ref,
             wc_ref, bc_ref, tab_o):
        m = (pe_ref[...] * (p0_ref[...] + p1_ref[...])
             + s_ref[...] * h1_ref[...])
        h2 = jnp.dot(m, w_ref[...], preferred_element_type=jnp.float32)
        h2 = h2 + b_ref[...]
        tab_o[...] = jnp.dot(h2, wc_ref[...],
                             preferred_element_type=jnp.float32) + bc_ref[...]

    rb = pl.BlockSpec((_R, 128), lambda i: (i, _Z))
    vb = pl.BlockSpec((_R, 1), lambda i: (i, _Z))
    return pl.pallas_call(
        body,
        grid=grid,
        in_specs=[rb, rb, rb, vb, vb,
                  pl.BlockSpec((128, 128), lambda i: (_Z, _Z)),
                  pl.BlockSpec((1, 128), lambda i: (_Z, _Z)),
                  pl.BlockSpec((128, 8), lambda i: (_Z, _Z)),
                  pl.BlockSpec((1, 8), lambda i: (_Z, _Z))],
        out_specs=pl.BlockSpec((_R, 8), lambda i: (i, _Z)),
        out_shape=jax.ShapeDtypeStruct((n, 8), jnp.float32),
    )(h1, p0, p1, pe2, s2, w2t, b2, wct, bc)


def _impl(x, edge_index, pan_w1, lin_W1, lin_b1, pan_w2, lin_W2, lin_b2,
          cls_W, cls_b):
    n = x.shape[0]
    e = edge_index.shape[1]
    hid = lin_W1.shape[0]
    x = x.astype(jnp.float32)

    nw = NC * NS
    ep_w = _ceil_to(-(-e // nw), 256)
    ep = ep_w * nw
    n_chunks = ep_w // CH
    rpt = _ceil_to(-(-(n + 1) // NS), CH)
    npad = rpt * NS

    src = edge_index[0].astype(jnp.int32)
    dst = edge_index[1].astype(jnp.int32)
    pad = ep - e
    pidx = lax.iota(jnp.int32, pad)
    spare = npad - n
    srcp = jnp.concatenate([src, pidx % jnp.int32(n)])
    dst_sc = jnp.concatenate([dst, jnp.int32(n) + pidx % jnp.int32(spare)])
    dst_ga = jnp.concatenate([dst, jnp.zeros((pad,), jnp.int32)])

    d1 = pan_w1[0].astype(jnp.float32)
    e1 = (pan_w1[0] * pan_w1[1]).astype(jnp.float32)
    d2 = pan_w2[0].astype(jnp.float32)
    e2 = (pan_w2[0] * pan_w2[1]).astype(jnp.float32)
    cvec = jnp.stack([d1, e1, d2, e2])

    degp = _sc_degree(dst_sc, npad, n_chunks)
    dp0 = lax.slice(degp, (0, 0), (n, 128))
    dp1 = lax.slice(degp, (npad, 0), (npad + n, 128))

    xs1, pe1, s1, dinv2, pe2, s2 = _tc_prescale(cvec, x, dp0, dp1)

    agg1 = _sc_spmm(xs1, srcp, dst_sc, npad, n_chunks)
    a10 = lax.slice(agg1, (0, 0), (n, hid))
    a11 = lax.slice(agg1, (npad, 0), (npad + n, hid))

    w1t = lin_W1.astype(jnp.float32).T
    b1 = lin_b1.astype(jnp.float32).reshape(1, hid)
    h1, xs2 = _tc_layer1(x, a10, a11, pe1, s1, dinv2, w1t, b1)

    agg2 = _sc_spmm(xs2, srcp, dst_sc, npad, n_chunks)
    a20 = lax.slice(agg2, (0, 0), (n, hid))
    a21 = lax.slice(agg2, (npad, 0), (npad + n, hid))

    w2t = lin_W2.astype(jnp.float32).T
    b2 = lin_b2.astype(jnp.float32).reshape(1, hid)
    out_dim = cls_W.shape[0]
    wa = lax.slice(cls_W, (0, 0), (out_dim, hid))
    wb = lax.slice(cls_W, (0, hid), (out_dim, 2 * hid))
    wct = jnp.concatenate([wa, wb], axis=0).astype(jnp.float32).T
    bc = jnp.concatenate([jnp.zeros((out_dim,), jnp.float32),
                          cls_b.astype(jnp.float32)]).reshape(1, 2 * out_dim)
    tab = _tc_layer2_cls(h1, a20, a21, pe2, s2, w2t, b2, wct, bc)

    npack = _ceil_to(n, 16)
    if npack != n:
        tab = jnp.concatenate(
            [tab, jnp.zeros((npack - n, 2 * out_dim), jnp.float32)])
    ttab = tab.reshape(npack * 8 // 128, 128)
    o = _sc_edge_combine(ttab, srcp, dst_ga, ep_w)
    return o.reshape(ep, 4)[:e]


kernel = jax.jit(_impl)


# speedup vs baseline: 13.3848x; 1.0391x over previous
import functools

import jax
import jax.numpy as jnp
import numpy as np
from jax import lax
from jax.experimental import pallas as pl
from jax.experimental.pallas import tpu as pltpu
from jax.experimental.pallas import tpu_sc as plsc

NC = 2
NS = 16
CH = 128


def _ceil_to(a, m):
    return (a + m - 1) // m * m


def _i32(v):
    return lax.convert_element_type(v, jnp.int32)


def _sc_degree(dstp, npad, n_chunks):
    rpt = npad // NS
    mesh = plsc.VectorSubcoreMesh(core_axis_name="c", subcore_axis_name="s")
    ep_w = n_chunks * CH
    NB = 2
    assert n_chunks % NB == 0

    @functools.partial(
        pl.kernel,
        out_type=jax.ShapeDtypeStruct((NC * npad, 128), jnp.float32),
        mesh=mesh,
        scratch_types=(
            [pltpu.VMEM_SHARED((npad, 128), jnp.float32),
             pltpu.VMEM((CH, 128), jnp.float32)]
            + [pltpu.VMEM((CH,), jnp.int32)] * NB
            + [pltpu.SemaphoreType.DMA] * NB
        ),
    )
    def k(dst_hbm, out_hbm, acc, buf, *scr):
        didx = scr[0:NB]
        sems = scr[NB:2 * NB]
        c = _i32(lax.axis_index("c"))
        s = _i32(lax.axis_index("s"))
        wid = s * jnp.int32(NC) + c
        srow = s * jnp.int32(rpt)

        def zrow(i, _):
            for j in range(8):
                buf[i, pl.ds(j * 16, 16)] = jnp.zeros((16,), jnp.float32)
            return 0
        lax.fori_loop(jnp.int32(0), jnp.int32(CH), zrow, 0)
        for b in range(rpt // CH):
            pltpu.sync_copy(buf, acc.at[pl.ds(srow + jnp.int32(b * CH), CH)])
        plsc.subcore_barrier()

        def orow(i, _):
            for j in range(8):
                buf[i, pl.ds(j * 16, 16)] = jnp.ones((16,), jnp.float32)
            return 0
        lax.fori_loop(jnp.int32(0), jnp.int32(CH), orow, 0)

        for b in range(NB):
            base = wid * jnp.int32(ep_w) + jnp.int32(b * CH)
            pltpu.async_copy(dst_hbm.at[pl.ds(base, CH)], didx[b], sems[b])

        def body(g, _):
            for b in range(NB):
                kk = g * jnp.int32(NB) + jnp.int32(b)
                base = wid * jnp.int32(ep_w) + kk * jnp.int32(CH)
                pltpu.make_async_copy(dst_hbm.at[pl.ds(base, CH)], didx[b],
                                      sems[b]).wait()
                pltpu.sync_copy(buf, acc.at[didx[b]], add=True)
                kn = kk + jnp.int32(NB)

                @pl.when(kn < jnp.int32(n_chunks))
                def _():
                    nbase = wid * jnp.int32(ep_w) + kn * jnp.int32(CH)
                    pltpu.async_copy(dst_hbm.at[pl.ds(nbase, CH)], didx[b],
                                     sems[b])
            return 0
        lax.fori_loop(jnp.int32(0), jnp.int32(n_chunks // NB), body, 0)
        plsc.subcore_barrier()
        pltpu.sync_copy(acc.at[pl.ds(srow, rpt)],
                        out_hbm.at[pl.ds(c * jnp.int32(npad) + srow, rpt)])

    return k(dstp)


def _sc_spmm(xs, srcp, dstp, npad, n_chunks):
    d = xs.shape[1]
    rpt = npad // NS
    ep_w = n_chunks * CH
    NB = 2
    assert n_chunks % NB == 0
    mesh = plsc.VectorSubcoreMesh(core_axis_name="c", subcore_axis_name="s")

    @functools.partial(
        pl.kernel,
        out_type=jax.ShapeDtypeStruct((NC * npad, d), jnp.float32),
        mesh=mesh,
        scratch_types=(
            [pltpu.VMEM_SHARED((npad, d), jnp.float32)]
            + [pltpu.VMEM((CH, d), jnp.float32)] * NB
            + [pltpu.VMEM((CH,), jnp.int32)] * (2 * NB)
            + [pltpu.SemaphoreType.DMA] * NB
        ),
    )
    def k(xs_hbm, src_hbm, dst_hbm, out_hbm, acc, *scr):
        rows = scr[0:NB]
        sidx = scr[NB:2 * NB]
        didx = scr[2 * NB:3 * NB]
        sems = scr[3 * NB:4 * NB]
        c = _i32(lax.axis_index("c"))
        s = _i32(lax.axis_index("s"))
        wid = s * jnp.int32(NC) + c
        srow = s * jnp.int32(rpt)

        def zrow(i, _):
            for j in range(d // 16):
                rows[0][i, pl.ds(j * 16, 16)] = jnp.zeros((16,), jnp.float32)
            return 0
        lax.fori_loop(jnp.int32(0), jnp.int32(CH), zrow, 0)
        for b in range(rpt // CH):
            pltpu.sync_copy(rows[0],
                            acc.at[pl.ds(srow + jnp.int32(b * CH), CH)])
        plsc.subcore_barrier()

        for b in range(NB):
            base = wid * jnp.int32(ep_w) + jnp.int32(b * CH)
            pltpu.sync_copy(src_hbm.at[pl.ds(base, CH)], sidx[b])
            pltpu.sync_copy(dst_hbm.at[pl.ds(base, CH)], didx[b])
            pltpu.async_copy(xs_hbm.at[sidx[b]], rows[b], sems[b])

        def body(g, _):
            for b in range(NB):
                kk = g * jnp.int32(NB) + jnp.int32(b)
                pltpu.make_async_copy(xs_hbm.at[sidx[b]], rows[b],
                                      sems[b]).wait()
                pltpu.sync_copy(rows[b], acc.at[didx[b]], add=True)
                kn = kk + jnp.int32(NB)

                @pl.when(kn < jnp.int32(n_chunks))
                def _():
                    base = wid * jnp.int32(ep_w) + kn * jnp.int32(CH)
                    pltpu.sync_copy(src_hbm.at[pl.ds(base, CH)], sidx[b])
                    pltpu.sync_copy(dst_hbm.at[pl.ds(base, CH)], didx[b])
                    pltpu.async_copy(xs_hbm.at[sidx[b]], rows[b], sems[b])
            return 0
        lax.fori_loop(jnp.int32(0), jnp.int32(n_chunks // NB), body, 0)
        plsc.subcore_barrier()
        pltpu.sync_copy(acc.at[pl.ds(srow, rpt)],
                        out_hbm.at[pl.ds(c * jnp.int32(npad) + srow, rpt)])

    return k(xs, srcp, dstp)


def _sc_edge_combine(ttab, srcp, dstp, ep_w):
    ep = srcp.shape[0]
    trows = ttab.shape[0]
    EC = 256
    n2 = ep_w // EC
    rows_w = ep_w * 4 // 128
    mesh = plsc.VectorSubcoreMesh(core_axis_name="c", subcore_axis_name="s")

    @functools.partial(
        pl.kernel,
        out_type=jax.ShapeDtypeStruct((ep * 4 // 128, 128), jnp.float32),
        mesh=mesh,
        compiler_params=pltpu.CompilerParams(needs_layout_passes=False),
        scratch_types=[
            pltpu.VMEM((trows, 128), jnp.float32),
            pltpu.VMEM((EC * 4 // 128, 128), jnp.float32),
            pltpu.VMEM((EC,), jnp.int32),
            pltpu.VMEM((EC,), jnp.int32),
        ],
    )
    def k(tab_hbm, src_hbm, dst_hbm, out_hbm, tbuf, obuf, sidx, didx):
        c = _i32(lax.axis_index("c"))
        s = _i32(lax.axis_index("s"))
        wid = s * jnp.int32(NC) + c
        pltpu.sync_copy(tab_hbm, tbuf)
        lane = lax.iota(jnp.int32, 16)

        def body(kk, _):
            base = wid * jnp.int32(ep_w) + kk * jnp.int32(EC)
            pltpu.sync_copy(src_hbm.at[pl.ds(base, EC)], sidx)
            pltpu.sync_copy(dst_hbm.at[pl.ds(base, EC)], didx)
            for j in range(EC // 16):
                sv = sidx[pl.ds(j * 16, 16)] * np.int32(8)
                dv = didx[pl.ds(j * 16, 16)] * np.int32(8) + np.int32(4)
                for cc in range(4):
                    fa = sv + np.int32(cc)
                    fb = dv + np.int32(cc)
                    a = plsc.load_gather(
                        tbuf, [lax.shift_right_logical(fa, np.int32(7)),
                               lax.bitwise_and(fa, np.int32(127))])
                    b = plsc.load_gather(
                        tbuf, [lax.shift_right_logical(fb, np.int32(7)),
                               lax.bitwise_and(fb, np.int32(127))])
                    fo = (lane + np.int32(j * 16)) * np.int32(4) + np.int32(cc)
                    plsc.store_scatter(
                        obuf, [lax.shift_right_logical(fo, np.int32(7)),
                               lax.bitwise_and(fo, np.int32(127))], a + b)
            rowbase = wid * jnp.int32(rows_w) + kk * jnp.int32(EC * 4 // 128)
            pltpu.sync_copy(obuf, out_hbm.at[pl.ds(rowbase, EC * 4 // 128)])
            return 0
        lax.fori_loop(jnp.int32(0), jnp.int32(n2), body, 0)

    return k(ttab, srcp, dstp)


_R = 256
_Z = np.int32(0)


def _tc_prescale(cvec, x, dp0, dp1):
    n = x.shape[0]
    grid = (pl.cdiv(n, _R),)

    def body(c_ref, x_ref, d0_ref, d1_ref,
             xs1_o, pe1_o, s1_o, dinv2_o, pe2_o, s2_o):
        deg_e = d0_ref[...] + d1_ref[...]
        d1 = c_ref[0]
        e1 = c_ref[1]
        d2 = c_ref[2]
        e2 = c_ref[3]
        g1 = d1 + e1 * deg_e
        g2 = d2 + e2 * deg_e
        i1 = jnp.where(g1 > 0, lax.rsqrt(g1), 0.0)
        i2 = jnp.where(g2 > 0, lax.rsqrt(g2), 0.0)
        xs1_o[...] = i1 * x_ref[...]
        pe1_o[...] = e1 * i1
        s1_o[...] = d1 * i1 * i1
        dinv2_o[...] = i2
        pe2_o[...] = e2 * i2
        s2_o[...] = d2 * i2 * i2

    v1 = jax.ShapeDtypeStruct((n, 1), jnp.float32)
    return pl.pallas_call(
        body,
        grid=grid,
        in_specs=[
            pl.BlockSpec((4,), lambda i: (_Z,), memory_space=pltpu.SMEM),
            pl.BlockSpec((_R, 128), lambda i: (i, _Z)),
            pl.BlockSpec((_R, 1), lambda i: (i, _Z)),
            pl.BlockSpec((_R, 1), lambda i: (i, _Z)),
        ],
        out_specs=[
            pl.BlockSpec((_R, 128), lambda i: (i, _Z)),
            pl.BlockSpec((_R, 1), lambda i: (i, _Z)),
            pl.BlockSpec((_R, 1), lambda i: (i, _Z)),
            pl.BlockSpec((_R, 1), lambda i: (i, _Z)),
            pl.BlockSpec((_R, 1), lambda i: (i, _Z)),
            pl.BlockSpec((_R, 1), lambda i: (i, _Z)),
        ],
        out_shape=(jax.ShapeDtypeStruct((n, 128), jnp.float32),
                   v1, v1, v1, v1, v1),
    )(cvec, x, dp0, dp1)


def _tc_layer1(x, p0, p1, pe1, s1, dinv2, w1t, b1):
    n = x.shape[0]
    grid = (pl.cdiv(n, _R),)

    def body(x_ref, p0_ref, p1_ref, pe_ref, s_ref, di_ref, w_ref, b_ref,
             h_o, xs_o):
        m = pe_ref[...] * (p0_ref[...] + p1_ref[...]) + s_ref[...] * x_ref[...]
        h = jnp.dot(m, w_ref[...], preferred_element_type=jnp.float32)
        h = jnp.maximum(h + b_ref[...], 0.0)
        h_o[...] = h
        xs_o[...] = di_ref[...] * h

    rb = pl.BlockSpec((_R, 128), lambda i: (i, _Z))
    vb = pl.BlockSpec((_R, 1), lambda i: (i, _Z))
    return pl.pallas_call(
        body,
        grid=grid,
        in_specs=[rb, rb, rb, vb, vb, vb,
                  pl.BlockSpec((128, 128), lambda i: (_Z, _Z)),
                  pl.BlockSpec((1, 128), lambda i: (_Z, _Z))],
        out_specs=[rb, rb],
        out_shape=(jax.ShapeDtypeStruct((n, 128), jnp.float32),
                   jax.ShapeDtypeStruct((n, 128), jnp.float32)),
    )(x, p0, p1, pe1, s1, dinv2, w1t, b1)


def _tc_layer2_cls(h1, p0, p1, pe2, s2, w2t, b2, wct, bc):
    n = h1.shape[0]
    grid = (pl.cdiv(n, _R),)

    def body(h1_ref, p0_ref, p1_ref, pe_ref, s_ref, w_ref, b_ref,
             wc_ref, bc_ref, tab_o):
        m = (pe_ref[...] * (p0_ref[...] + p1_ref[...])
             + s_ref[...] * h1_ref[...])
        h2 = jnp.dot(m, w_ref[...], preferred_element_type=jnp.float32)
        h2 = h2 + b_ref[...]
        tab_o[...] = jnp.dot(h2, wc_ref[...],
                             preferred_element_type=jnp.float32) + bc_ref[...]

    rb = pl.BlockSpec((_R, 128), lambda i: (i, _Z))
    vb = pl.BlockSpec((_R, 1), lambda i: (i, _Z))
    return pl.pallas_call(
        body,
        grid=grid,
        in_specs=[rb, rb, rb, vb, vb,
                  pl.BlockSpec((128, 128), lambda i: (_Z, _Z)),
                  pl.BlockSpec((1, 128), lambda i: (_Z, _Z)),
                  pl.BlockSpec((128, 8), lambda i: (_Z, _Z)),
                  pl.BlockSpec((1, 8), lambda i: (_Z, _Z))],
        out_specs=pl.BlockSpec((_R, 8), lambda i: (i, _Z)),
        out_shape=jax.ShapeDtypeStruct((n, 8), jnp.float32),
    )(h1, p0, p1, pe2, s2, w2t, b2, wct, bc)


def _impl(x, edge_index, pan_w1, lin_W1, lin_b1, pan_w2, lin_W2, lin_b2,
          cls_W, cls_b):
    n = x.shape[0]
    e = edge_index.shape[1]
    hid = lin_W1.shape[0]
    x = x.astype(jnp.float32)

    nw = NC * NS
    ep_w = _ceil_to(-(-e // nw), 256)
    ep = ep_w * nw
    n_chunks = ep_w // CH
    rpt = _ceil_to(-(-(n + 1) // NS), CH)
    npad = rpt * NS

    src = edge_index[0].astype(jnp.int32)
    dst = edge_index[1].astype(jnp.int32)
    pad = ep - e
    pidx = lax.iota(jnp.int32, pad)
    spare = npad - n
    srcp = jnp.concatenate([src, pidx % jnp.int32(n)])
    dst_sc = jnp.concatenate([dst, jnp.int32(n) + pidx % jnp.int32(spare)])
    dst_ga = jnp.concatenate([dst, jnp.zeros((pad,), jnp.int32)])

    d1 = pan_w1[0].astype(jnp.float32)
    e1 = (pan_w1[0] * pan_w1[1]).astype(jnp.float32)
    d2 = pan_w2[0].astype(jnp.float32)
    e2 = (pan_w2[0] * pan_w2[1]).astype(jnp.float32)
    cvec = jnp.stack([d1, e1, d2, e2])

    degp = _sc_degree(dst_sc, npad, n_chunks)
    dp0 = lax.slice(degp, (0, 0), (n, 1))
    dp1 = lax.slice(degp, (npad, 0), (npad + n, 1))

    xs1, pe1, s1, dinv2, pe2, s2 = _tc_prescale(cvec, x, dp0, dp1)

    agg1 = _sc_spmm(xs1, srcp, dst_sc, npad, n_chunks)
    a10 = lax.slice(agg1, (0, 0), (n, hid))
    a11 = lax.slice(agg1, (npad, 0), (npad + n, hid))

    w1t = lin_W1.astype(jnp.float32).T
    b1 = lin_b1.astype(jnp.float32).reshape(1, hid)
    h1, xs2 = _tc_layer1(x, a10, a11, pe1, s1, dinv2, w1t, b1)

    agg2 = _sc_spmm(xs2, srcp, dst_sc, npad, n_chunks)
    a20 = lax.slice(agg2, (0, 0), (n, hid))
    a21 = lax.slice(agg2, (npad, 0), (npad + n, hid))

    w2t = lin_W2.astype(jnp.float32).T
    b2 = lin_b2.astype(jnp.float32).reshape(1, hid)
    out_dim = cls_W.shape[0]
    wa = lax.slice(cls_W, (0, 0), (out_dim, hid))
    wb = lax.slice(cls_W, (0, hid), (out_dim, 2 * hid))
    wct = jnp.concatenate([wa, wb], axis=0).astype(jnp.float32).T
    bc = jnp.concatenate([jnp.zeros((out_dim,), jnp.float32),
                          cls_b.astype(jnp.float32)]).reshape(1, 2 * out_dim)
    tab = _tc_layer2_cls(h1, a20, a21, pe2, s2, w2t, b2, wct, bc)

    npack = _ceil_to(n, 16)
    if npack != n:
        tab = jnp.concatenate(
            [tab, jnp.zeros((npack - n, 2 * out_dim), jnp.float32)])
    ttab = tab.reshape(npack * 8 // 128, 128)
    o = _sc_edge_combine(ttab, srcp, dst_ga, ep_w)
    return o.reshape(ep, 4)[:e]


kernel = jax.jit(_impl)
